# Initial kernel scaffold; baseline (speedup 1.0000x reference)
import jax
import jax.numpy as jnp
from jax.experimental import pallas as pl


def kernel(hidden_states, Wg, W1, W2, W3):
    raise NotImplementedError("write your pallas kernel here")


# speedup vs baseline: 1.4694x; 1.4694x over previous
import functools

import jax
import jax.numpy as jnp
from jax import lax
from jax.experimental import pallas as pl
from jax.experimental.pallas import tpu as pltpu
from jax.experimental.pallas import tpu_sc as plsc

E = 8
H = 1024
F = 3584
T = 2048
BLK = 256
NB = 24
PAD = NB * BLK
FCH = 512
NF = F // FCH
NC = 2
NS = 16
NW = NC * NS
TPW = T // NW
CH = 32

_F32 = jnp.float32
_I32 = jnp.int32


def _router_body(x_ref, wg_ref, meta_ref, pos0_ref, pos1_ref, w0_ref, w1_ref):
    x = x_ref[...]
    wg = wg_ref[...]
    logits = lax.dot_general(x, wg, (((1,), (1,)), ((), ())),
                             preferred_element_type=_F32)
    m = jnp.max(logits, axis=1, keepdims=True)
    ex = jnp.exp(logits - m)
    probs = ex / jnp.sum(ex, axis=1, keepdims=True)

    lane = lax.broadcasted_iota(_I32, (T, E), 1).astype(_F32)
    m1 = jnp.max(probs, axis=1, keepdims=True)
    e1 = jnp.min(jnp.where(probs == m1, lane, float(E)), axis=1, keepdims=True)
    one0 = (lane == e1).astype(_F32)
    probs2 = jnp.where(lane == e1, -1.0, probs)
    m2 = jnp.max(probs2, axis=1, keepdims=True)
    e2 = jnp.min(jnp.where(probs2 == m2, lane, float(E)), axis=1, keepdims=True)
    one1 = (lane == e2).astype(_F32)

    cat = jnp.concatenate([one0, one1], axis=1)
    s = cat
    sh = 1
    while sh < T:
        z = jnp.concatenate([jnp.zeros((sh, 2 * E), _F32), s[: T - sh]], axis=0)
        s = s + z
        sh *= 2
    sx = s - cat
    tot = s[T - 1 : T, :]
    cnt0 = tot[:, :E]
    counts = cnt0 + tot[:, E:]

    rank0 = jnp.sum(sx[:, :E] * one0, axis=1, keepdims=True)
    rank1 = (jnp.sum(sx[:, E:] * one1, axis=1, keepdims=True)
             + jnp.sum(one1 * cnt0, axis=1, keepdims=True))

    pcb = jnp.floor((counts + float(BLK - 1)) / float(BLK))
    p = pcb
    sh = 1
    while sh < E:
        p = p + jnp.concatenate([jnp.zeros((1, sh), _F32), p[:, : E - sh]], axis=1)
        sh *= 2
    nbc = p
    po = (nbc - pcb) * float(BLK)

    pos0 = jnp.sum(one0 * po, axis=1, keepdims=True) + rank0
    pos1 = jnp.sum(one1 * po, axis=1, keepdims=True) + rank1

    ident = (lax.broadcasted_iota(_I32, (E, E), 0)
             == lax.broadcasted_iota(_I32, (E, E), 1)).astype(_F32)
    nbc_col = lax.dot_general(ident, nbc, (((1,), (1,)), ((), ())),
                              preferred_element_type=_F32)
    bcols = lax.broadcasted_iota(_I32, (E, NB), 1).astype(_F32)
    be_row = jnp.sum((nbc_col <= bcols).astype(_F32), axis=0, keepdims=True)
    used = nbc[:, E - 1 : E]
    brow = lax.broadcasted_iota(_I32, (1, NB), 1).astype(_F32)
    be_last = jnp.sum(jnp.where(brow == used - 1.0, be_row, 0.0),
                      axis=1, keepdims=True)
    be_final = jnp.where(brow < used, be_row, be_last)
    xclamp = jnp.minimum(brow, used - 1.0)

    meta = jnp.concatenate(
        [be_final, xclamp, used,
         jnp.zeros((1, 128 - 2 * NB - 1), _F32)], axis=1).astype(_I32)
    meta_ref[...] = meta
    pos0_ref[...] = pos0.astype(_I32)
    pos1_ref[...] = pos1.astype(_I32)
    w0_ref[...] = m1
    w1_ref[...] = m2


def _router(x, wg):
    out_shapes = (
        jax.ShapeDtypeStruct((1, 128), _I32),
        jax.ShapeDtypeStruct((T, 1), _I32),
        jax.ShapeDtypeStruct((T, 1), _I32),
        jax.ShapeDtypeStruct((T, 1), _F32),
        jax.ShapeDtypeStruct((T, 1), _F32),
    )
    return pl.pallas_call(_router_body, out_shape=out_shapes)(x, wg)


def _ffn_body(meta_ref, xs_ref, w1_ref, w3_ref, w2_ref, out_ref):
    b = pl.program_id(0)
    f = pl.program_id(1)
    used = meta_ref[0, 2 * NB]

    @pl.when(b < used)
    def _():
        x = xs_ref[...]
        a = lax.dot_general(x, w1_ref[0], (((1,), (1,)), ((), ())),
                            preferred_element_type=_F32)
        c = lax.dot_general(x, w3_ref[0], (((1,), (1,)), ((), ())),
                            preferred_element_type=_F32)
        g = (a / (1.0 + jnp.exp(-a))) * c
        contrib = lax.dot_general(g, w2_ref[0], (((1,), (1,)), ((), ())),
                                  preferred_element_type=_F32)

        @pl.when(f == 0)
        def _():
            out_ref[...] = contrib

        @pl.when(f != 0)
        def _():
            out_ref[...] += contrib


def _ffn(meta, xs, w1, w3, w2):
    grid_spec = pltpu.PrefetchScalarGridSpec(
        num_scalar_prefetch=1,
        grid=(NB, NF),
        in_specs=[
            pl.BlockSpec((BLK, H), lambda b, f, md: (md[0, NB + b], 0)),
            pl.BlockSpec((1, FCH, H), lambda b, f, md: (md[0, b], f, 0)),
            pl.BlockSpec((1, FCH, H), lambda b, f, md: (md[0, b], f, 0)),
            pl.BlockSpec((1, H, FCH), lambda b, f, md: (md[0, b], 0, f)),
        ],
        out_specs=pl.BlockSpec((BLK, H), lambda b, f, md: (md[0, NB + b], 0)),
    )
    return pl.pallas_call(
        _ffn_body,
        grid_spec=grid_spec,
        out_shape=jax.ShapeDtypeStruct((PAD, H), _F32),
        compiler_params=pltpu.CompilerParams(
            dimension_semantics=("arbitrary", "arbitrary")),
    )(meta, xs, w1, w3, w2)


def _dispatch(x, pos0, pos1):
    mesh = plsc.VectorSubcoreMesh(core_axis_name="c", subcore_axis_name="s",
                                  num_cores=NC, num_subcores=NS)

    @functools.partial(
        pl.kernel,
        out_type=jax.ShapeDtypeStruct((PAD, H), _F32),
        mesh=mesh,
        scratch_types=[
            pltpu.VMEM((TPW,), _I32),
            pltpu.VMEM((TPW,), _I32),
            pltpu.VMEM((TPW, H), _F32),
            pltpu.SemaphoreType.DMA,
            pltpu.SemaphoreType.DMA,
        ],
    )
    def body(x_hbm, p0_hbm, p1_hbm, xs_hbm, p0_v, p1_v, xbuf, sem0, sem1):
        wid = lax.axis_index("s") * NC + lax.axis_index("c")
        base = wid * TPW
        pltpu.sync_copy(p0_hbm.at[pl.ds(base, TPW)], p0_v)
        pltpu.sync_copy(p1_hbm.at[pl.ds(base, TPW)], p1_v)
        pltpu.sync_copy(x_hbm.at[pl.ds(base, TPW)], xbuf)
        d0 = pltpu.async_copy(xbuf, xs_hbm.at[p0_v], sem0)
        d1 = pltpu.async_copy(xbuf, xs_hbm.at[p1_v], sem1)
        d0.wait()
        d1.wait()

    return body(x, pos0, pos1)


def _combine(y, pos0, pos1, w0, w1):
    mesh = plsc.VectorSubcoreMesh(core_axis_name="c", subcore_axis_name="s",
                                  num_cores=NC, num_subcores=NS)

    @functools.partial(
        pl.kernel,
        out_type=jax.ShapeDtypeStruct((T, H), _F32),
        mesh=mesh,
        scratch_types=[
            pltpu.VMEM((CH,), _I32),
            pltpu.VMEM((CH,), _I32),
            pltpu.VMEM((CH,), _F32),
            pltpu.VMEM((CH,), _F32),
            pltpu.VMEM((CH, H), _F32),
            pltpu.VMEM((CH, H), _F32),
            pltpu.VMEM((CH, H), _F32),
            pltpu.SemaphoreType.DMA,
            pltpu.SemaphoreType.DMA,
        ],
    )
    def body(y_hbm, p0_hbm, p1_hbm, w0_hbm, w1_hbm, out_hbm,
             p0_v, p1_v, w0_v, w1_v, y0buf, y1buf, obuf, sem0, sem1):
        wid = lax.axis_index("s") * NC + lax.axis_index("c")
        for chunk in range(TPW // CH):
            base = wid * TPW + chunk * CH
            pltpu.sync_copy(p0_hbm.at[pl.ds(base, CH)], p0_v)
            pltpu.sync_copy(p1_hbm.at[pl.ds(base, CH)], p1_v)
            pltpu.sync_copy(w0_hbm.at[pl.ds(base, CH)], w0_v)
            pltpu.sync_copy(w1_hbm.at[pl.ds(base, CH)], w1_v)
            d0 = pltpu.async_copy(y_hbm.at[p0_v], y0buf, sem0)
            d1 = pltpu.async_copy(y_hbm.at[p1_v], y1buf, sem1)
            d0.wait()
            d1.wait()

            for half in range(CH // 16):
                w0h = w0_v[pl.ds(half * 16, 16)]
                w1h = w1_v[pl.ds(half * 16, 16)]

                def row(i, _, w0h=w0h, w1h=w1h, half=half):
                    bidx = jnp.full((16,), i, _I32)
                    wa = w0h.at[bidx].get(mode="promise_in_bounds")
                    wb = w1h.at[bidx].get(mode="promise_in_bounds")
                    r = half * 16 + i

                    def col(j, _):
                        sl = pl.ds(j * 16, 16)
                        obuf[r, sl] = wa * y0buf[r, sl] + wb * y1buf[r, sl]
                        return 0

                    lax.fori_loop(0, H // 16, col, 0)
                    return 0

                lax.fori_loop(0, 16, row, 0)
            pltpu.sync_copy(obuf, out_hbm.at[pl.ds(base, CH)])

    return body(y, pos0, pos1, w0, w1)


def kernel(hidden_states, Wg, W1, W2, W3):
    b, s, h = hidden_states.shape
    x = hidden_states.reshape(T, H)
    meta, pos0, pos1, w0, w1 = _router(x, Wg)
    pos0 = pos0.reshape(T)
    pos1 = pos1.reshape(T)
    xs = _dispatch(x, pos0, pos1)
    y = _ffn(meta, xs, W1, W3, W2)
    out = _combine(y, pos0, pos1, w0.reshape(T), w1.reshape(T))
    return out.reshape(b, s, h)
